# baseline (speedup 1.0000x reference)
import functools

import jax
import jax.numpy as jnp
from jax import lax
from jax.experimental import pallas as pl
from jax.experimental.pallas import tpu as pltpu
from jax.experimental.pallas import tpu_sc as plsc

N = 10000
E = 320000
D = 128

NC = 2
NS = 16
LANES = 16
NW = NC * NS
CH = 80
NCHUNK = E // CH
CHUNKS_PER_W = (NCHUNK + NW - 1) // NW
ZROWS = 40
NZBLK = N // ZROWS
ZBLK_PER_S = (NZBLK + NS - 1) // NS


def _edge_proj(efeat, Ws, bs):
    BE = 2560
    grid = (E // BE,)
    nl = Ws.shape[0]

    def body(x_ref, w_ref, b_ref, *outs):
        x = x_ref[...].astype(jnp.bfloat16)
        for i in range(nl):
            acc = jnp.dot(x, w_ref[i].astype(jnp.bfloat16),
                          preferred_element_type=jnp.float32)
            outs[i][...] = acc + b_ref[i][None, :]

    return pl.pallas_call(
        body,
        grid=grid,
        in_specs=[
            pl.BlockSpec((BE, D), lambda i: (i, 0)),
            pl.BlockSpec((nl, D, D), lambda i: (0, 0, 0)),
            pl.BlockSpec((nl, D), lambda i: (0, 0)),
        ],
        out_specs=[pl.BlockSpec((BE, D), lambda i: (i, 0))] * nl,
        out_shape=[jax.ShapeDtypeStruct((E, D), jnp.float32)] * nl,
    )(efeat, Ws, bs)


def _sc_edge_agg(h, e, src, dst):
    mesh = plsc.VectorSubcoreMesh(
        core_axis_name="c", subcore_axis_name="s",
        num_cores=NC, num_subcores=NS)

    NB = 3

    @functools.partial(
        pl.kernel,
        out_type=jax.ShapeDtypeStruct((NC, N, D), jnp.float32),
        mesh=mesh,
        scratch_types=[
            [pltpu.VMEM((CH,), jnp.int32) for _ in range(NB)],
            [pltpu.VMEM((CH,), jnp.int32) for _ in range(NB)],
            [pltpu.VMEM((CH, D), jnp.float32) for _ in range(NB)],
            pltpu.VMEM((ZROWS, D), jnp.float32),
            pltpu.VMEM_SHARED((N, D), jnp.float32),
            [pltpu.SemaphoreType.DMA for _ in range(NB)],
            [pltpu.SemaphoreType.DMA for _ in range(NB)],
            [pltpu.SemaphoreType.DMA for _ in range(NB)],
        ],
    )
    def k(h_hbm, e_hbm, src_hbm, dst_hbm, out_hbm,
          sidx, didx, msg, zbuf, acc, semL, semG, semS):
        cid = lax.axis_index("c")
        sid = lax.axis_index("s")
        wid = sid * NC + cid
        jmax = (NCHUNK - wid + NW - 1) // NW

        def chunk_valid(j):
            return j < jmax

        def issue_loads(j, b):
            base = (wid + j * NW) * CH
            pltpu.async_copy(src_hbm.at[pl.ds(base, CH)], sidx[b], semL[b])
            pltpu.async_copy(dst_hbm.at[pl.ds(base, CH)], didx[b], semL[b])
            pltpu.async_copy(e_hbm.at[pl.ds(base, CH)], msg[b], semL[b])

        def wait_loads(b):
            pltpu.make_async_copy(src_hbm.at[pl.ds(0, CH)], sidx[b], semL[b]).wait()
            pltpu.make_async_copy(dst_hbm.at[pl.ds(0, CH)], didx[b], semL[b]).wait()
            pltpu.make_async_copy(e_hbm.at[pl.ds(0, CH)], msg[b], semL[b]).wait()

        def zrow(r, carry):
            for j in range(D // LANES):
                zbuf[r, pl.ds(j * LANES, LANES)] = jnp.zeros((LANES,), jnp.float32)
            return carry
        lax.fori_loop(0, ZROWS, zrow, 0)
        for t in range(ZBLK_PER_S):
            b = sid + t * NS

            @pl.when(b < NZBLK)
            def _():
                pltpu.sync_copy(zbuf, acc.at[pl.ds(b * ZROWS, ZROWS)])
        plsc.subcore_barrier()

        issue_loads(0, 0)

        NROUND = (CHUNKS_PER_W + 2 + NB - 1) // NB + 1

        def round_body(r, carry):
            for b in range(NB):
                j = r * NB + b

                @pl.when(chunk_valid(j))
                def _():
                    wait_loads(b)
                    pltpu.async_copy(h_hbm.at[sidx[b]], msg[b], semG[b], add=True)

                bn = (b + 1) % NB

                @pl.when((j >= 2) & chunk_valid(j - 2))
                def _():
                    pltpu.make_async_copy(msg[bn], acc.at[didx[bn]], semS[bn]).wait()

                @pl.when((j + 1 >= 1) & chunk_valid(j + 1))
                def _():
                    issue_loads(j + 1, bn)

                @pl.when(chunk_valid(j))
                def _():
                    pltpu.make_async_copy(h_hbm.at[sidx[b]], msg[b], semG[b]).wait()

                    @plsc.parallel_loop(0, CH, 1, unroll=4)
                    def _relu_row(r):
                        for q in range(D // LANES):
                            sl = pl.ds(q * LANES, LANES)
                            msg[b][r, sl] = jnp.maximum(msg[b][r, sl], 0.0)
                    pltpu.async_copy(msg[b], acc.at[didx[b]], semS[b], add=True)
            return carry
        lax.fori_loop(0, NROUND, round_body, 0)

        plsc.subcore_barrier()
        for t in range(ZBLK_PER_S):
            b = sid + t * NS

            @pl.when(b < NZBLK)
            def _():
                pltpu.sync_copy(acc.at[pl.ds(b * ZROWS, ZROWS)], zbuf)
                pltpu.sync_copy(zbuf, out_hbm.at[cid, pl.ds(b * ZROWS, ZROWS)])

    return k(h, e, src, dst)


def _mlp(h, parts, W1, b1, g, be, W2, b2, relu_out):
    def body(h_ref, p_ref, w1_ref, b1_ref, g_ref, be_ref, w2_ref, b2_ref, o_ref):
        r = h_ref[...] + p_ref[0] + p_ref[1]
        t = jnp.dot(r, w1_ref[...], preferred_element_type=jnp.float32) + b1_ref[...]
        m = jnp.mean(t, axis=0, keepdims=True)
        v = jnp.mean((t - m) ** 2, axis=0, keepdims=True)
        t = (t - m) / jnp.sqrt(v + 1e-5) * g_ref[...] + be_ref[...]
        t = jnp.dot(jnp.maximum(t, 0.0), w2_ref[...],
                    preferred_element_type=jnp.float32) + b2_ref[...]
        if relu_out:
            t = jnp.maximum(t, 0.0)
        o_ref[...] = t

    return pl.pallas_call(
        body,
        out_shape=jax.ShapeDtypeStruct((N, D), jnp.float32),
    )(h, parts, W1, b1[None], g[None], be[None], W2, b2[None])


def kernel(nfeat, efeat, params, edge_index):
    src = edge_index[0]
    dst = edge_index[1]
    (e0,) = _edge_proj(efeat,
                       params["edge"][0][0][None], params["edge"][0][1][None])
    e1, e2 = _edge_proj(
        efeat,
        jnp.stack([params["edge"][1][0], params["edge"][2][0]]),
        jnp.stack([params["edge"][1][1], params["edge"][2][1]]))
    es = (e0, e1, e2)
    h = nfeat
    for i in range(3):
        parts = _sc_edge_agg(h, es[i], src, dst)
        p = params["mlp"][i]
        h = _mlp(h, parts, p["W1"], p["b1"], p["g"], p["be"], p["W2"], p["b2"],
                 relu_out=(i != 2))
    return h


# speedup vs baseline: 1.0387x; 1.0387x over previous
import functools

import jax
import jax.numpy as jnp
from jax import lax
from jax.experimental import pallas as pl
from jax.experimental.pallas import tpu as pltpu
from jax.experimental.pallas import tpu_sc as plsc

N = 10000
E = 320000
D = 128

NC = 2
NS = 16
LANES = 16
NW = NC * NS
CH = 80
ZROWS = 40
NZBLK = N // ZROWS
ZBLK_PER_S = (NZBLK + NS - 1) // NS


def _edge_proj(efeat, Ws, bs, row0, nrows):
    BE = 3200
    grid = (nrows // BE,)
    nl = Ws.shape[0]
    blk0 = row0 // BE

    def body(x_ref, w_ref, b_ref, *outs):
        x = x_ref[...].astype(jnp.bfloat16)
        for i in range(nl):
            acc = jnp.dot(x, w_ref[i].astype(jnp.bfloat16),
                          preferred_element_type=jnp.float32)
            outs[i][...] = acc + b_ref[i][None, :]

    return pl.pallas_call(
        body,
        grid=grid,
        in_specs=[
            pl.BlockSpec((BE, D), lambda i: (blk0 + i, 0)),
            pl.BlockSpec((nl, D, D), lambda i: (0, 0, 0)),
            pl.BlockSpec((nl, D), lambda i: (0, 0)),
        ],
        out_specs=[pl.BlockSpec((BE, D), lambda i: (i, 0))] * nl,
        out_shape=[jax.ShapeDtypeStruct((nrows, D), jnp.float32)] * nl,
    )(efeat, Ws, bs)


def _sc_edge_agg(h, e, src, dst, eoff, ne):
    mesh = plsc.VectorSubcoreMesh(
        core_axis_name="c", subcore_axis_name="s",
        num_cores=NC, num_subcores=NS)

    NB = 3
    nchunk = ne // CH
    chunks_per_w = (nchunk + NW - 1) // NW

    @functools.partial(
        pl.kernel,
        out_type=jax.ShapeDtypeStruct((NC, N, D), jnp.float32),
        mesh=mesh,
        scratch_types=[
            [pltpu.VMEM((CH,), jnp.int32) for _ in range(NB)],
            [pltpu.VMEM((CH,), jnp.int32) for _ in range(NB)],
            [pltpu.VMEM((CH, D), jnp.float32) for _ in range(NB)],
            pltpu.VMEM((ZROWS, D), jnp.float32),
            pltpu.VMEM_SHARED((N, D), jnp.float32),
            [pltpu.SemaphoreType.DMA for _ in range(NB)],
            [pltpu.SemaphoreType.DMA for _ in range(NB)],
            [pltpu.SemaphoreType.DMA for _ in range(NB)],
        ],
    )
    def k(h_hbm, e_hbm, src_hbm, dst_hbm, out_hbm,
          sidx, didx, msg, zbuf, acc, semL, semG, semS):
        cid = lax.axis_index("c")
        sid = lax.axis_index("s")
        wid = sid * NC + cid
        jmax = (nchunk - wid + NW - 1) // NW

        def chunk_valid(j):
            return j < jmax

        def issue_loads(j, b):
            ebase = (wid + j * NW) * CH
            base = eoff + ebase
            pltpu.async_copy(src_hbm.at[pl.ds(base, CH)], sidx[b], semL[b])
            pltpu.async_copy(dst_hbm.at[pl.ds(base, CH)], didx[b], semL[b])
            pltpu.async_copy(e_hbm.at[pl.ds(ebase, CH)], msg[b], semL[b])

        def wait_loads(b):
            pltpu.make_async_copy(src_hbm.at[pl.ds(0, CH)], sidx[b], semL[b]).wait()
            pltpu.make_async_copy(dst_hbm.at[pl.ds(0, CH)], didx[b], semL[b]).wait()
            pltpu.make_async_copy(e_hbm.at[pl.ds(0, CH)], msg[b], semL[b]).wait()

        def zrow(r, carry):
            for j in range(D // LANES):
                zbuf[r, pl.ds(j * LANES, LANES)] = jnp.zeros((LANES,), jnp.float32)
            return carry
        lax.fori_loop(0, ZROWS, zrow, 0)
        for t in range(ZBLK_PER_S):
            b = sid + t * NS

            @pl.when(b < NZBLK)
            def _():
                pltpu.sync_copy(zbuf, acc.at[pl.ds(b * ZROWS, ZROWS)])
        plsc.subcore_barrier()

        issue_loads(0, 0)

        nround = (chunks_per_w + 2 + NB - 1) // NB + 1

        def round_body(r, carry):
            for b in range(NB):
                j = r * NB + b

                @pl.when(chunk_valid(j))
                def _():
                    wait_loads(b)
                    pltpu.async_copy(h_hbm.at[sidx[b]], msg[b], semG[b], add=True)

                bn = (b + 1) % NB

                @pl.when((j >= 2) & chunk_valid(j - 2))
                def _():
                    pltpu.make_async_copy(msg[bn], acc.at[didx[bn]], semS[bn]).wait()

                @pl.when((j + 1 >= 1) & chunk_valid(j + 1))
                def _():
                    issue_loads(j + 1, bn)

                @pl.when(chunk_valid(j))
                def _():
                    pltpu.make_async_copy(h_hbm.at[sidx[b]], msg[b], semG[b]).wait()

                    @plsc.parallel_loop(0, CH, 1, unroll=4)
                    def _relu_row(r):
                        for q in range(D // LANES):
                            sl = pl.ds(q * LANES, LANES)
                            msg[b][r, sl] = jnp.maximum(msg[b][r, sl], 0.0)
                    pltpu.async_copy(msg[b], acc.at[didx[b]], semS[b], add=True)
            return carry
        lax.fori_loop(0, nround, round_body, 0)

        plsc.subcore_barrier()
        for t in range(ZBLK_PER_S):
            b = sid + t * NS

            @pl.when(b < NZBLK)
            def _():
                pltpu.sync_copy(acc.at[pl.ds(b * ZROWS, ZROWS)], zbuf)
                pltpu.sync_copy(zbuf, out_hbm.at[cid, pl.ds(b * ZROWS, ZROWS)])

    return k(h, e, src, dst)


def _mlp(h, parts_list, W1, b1, g, be, W2, b2, relu_out):
    np_ = len(parts_list)

    def body(h_ref, *refs):
        p_refs = refs[:np_]
        w1_ref, b1_ref, g_ref, be_ref, w2_ref, b2_ref, o_ref = refs[np_:]
        r = h_ref[...]
        for p_ref in p_refs:
            r = r + p_ref[0] + p_ref[1]
        t = jnp.dot(r, w1_ref[...], preferred_element_type=jnp.float32) + b1_ref[...]
        m = jnp.mean(t, axis=0, keepdims=True)
        v = jnp.mean((t - m) ** 2, axis=0, keepdims=True)
        t = (t - m) / jnp.sqrt(v + 1e-5) * g_ref[...] + be_ref[...]
        t = jnp.dot(jnp.maximum(t, 0.0), w2_ref[...],
                    preferred_element_type=jnp.float32) + b2_ref[...]
        if relu_out:
            t = jnp.maximum(t, 0.0)
        o_ref[...] = t

    return pl.pallas_call(
        body,
        out_shape=jax.ShapeDtypeStruct((N, D), jnp.float32),
    )(h, *parts_list, W1, b1[None], g[None], be[None], W2, b2[None])


def kernel(nfeat, efeat, params, edge_index):
    src = edge_index[0]
    dst = edge_index[1]
    E2 = E // 2
    (e0a,) = _edge_proj(efeat, params["edge"][0][0][None],
                        params["edge"][0][1][None], 0, E2)
    (e0b,) = _edge_proj(efeat, params["edge"][0][0][None],
                        params["edge"][0][1][None], E2, E2)
    e1, e2 = _edge_proj(
        efeat,
        jnp.stack([params["edge"][1][0], params["edge"][2][0]]),
        jnp.stack([params["edge"][1][1], params["edge"][2][1]]), 0, E)
    h = nfeat
    for i in range(3):
        if i == 0:
            parts = [_sc_edge_agg(h, e0a, src, dst, 0, E2),
                     _sc_edge_agg(h, e0b, src, dst, E2, E2)]
        else:
            parts = [_sc_edge_agg(h, (e1, e2)[i - 1], src, dst, 0, E)]
        p = params["mlp"][i]
        h = _mlp(h, parts, p["W1"], p["b1"], p["g"], p["be"], p["W2"], p["b2"],
                 relu_out=(i != 2))
    return h


# speedup vs baseline: 1.0531x; 1.0139x over previous
import functools

import jax
import jax.numpy as jnp
from jax import lax
from jax.experimental import pallas as pl
from jax.experimental.pallas import tpu as pltpu
from jax.experimental.pallas import tpu_sc as plsc

N = 10000
E = 320000
D = 128

NC = 2
NS = 16
LANES = 16
NW = NC * NS
CH = 80
ZROWS = 40
NZBLK = N // ZROWS
ZBLK_PER_S = (NZBLK + NS - 1) // NS


def _edge_proj(efeat, Ws, bs, row0, nrows):
    BE = 3200
    grid = (nrows // BE,)
    nl = Ws.shape[0]
    blk0 = row0 // BE

    def body(x_ref, w_ref, b_ref, *outs):
        x = x_ref[...].astype(jnp.bfloat16)
        for i in range(nl):
            acc = jnp.dot(x, w_ref[i].astype(jnp.bfloat16),
                          preferred_element_type=jnp.float32)
            outs[i][...] = acc + b_ref[i][None, :]

    return pl.pallas_call(
        body,
        grid=grid,
        in_specs=[
            pl.BlockSpec((BE, D), lambda i: (blk0 + i, 0)),
            pl.BlockSpec((nl, D, D), lambda i: (0, 0, 0)),
            pl.BlockSpec((nl, D), lambda i: (0, 0)),
        ],
        out_specs=[pl.BlockSpec((BE, D), lambda i: (i, 0))] * nl,
        out_shape=[jax.ShapeDtypeStruct((nrows, D), jnp.float32)] * nl,
    )(efeat, Ws, bs)


def _sc_edge_agg(h, e, src, dst, eoff, ne):
    mesh = plsc.VectorSubcoreMesh(
        core_axis_name="c", subcore_axis_name="s",
        num_cores=NC, num_subcores=NS)

    NB = 3
    nchunk = ne // CH
    chunks_per_w = (nchunk + NW - 1) // NW

    @functools.partial(
        pl.kernel,
        out_type=jax.ShapeDtypeStruct((NC, N, D), jnp.float32),
        mesh=mesh,
        scratch_types=[
            [pltpu.VMEM((CH,), jnp.int32) for _ in range(NB)],
            [pltpu.VMEM((CH,), jnp.int32) for _ in range(NB)],
            [pltpu.VMEM((CH, D), jnp.float32) for _ in range(NB)],
            [pltpu.VMEM((ZROWS, D), jnp.float32) for _ in range(2)],
            pltpu.VMEM_SHARED((N, D), jnp.float32),
            [pltpu.SemaphoreType.DMA for _ in range(NB)],
            [pltpu.SemaphoreType.DMA for _ in range(NB)],
            [pltpu.SemaphoreType.DMA for _ in range(NB)],
            pltpu.SemaphoreType.DMA,
            [pltpu.SemaphoreType.DMA for _ in range(2)],
        ],
    )
    def k(h_hbm, e_hbm, src_hbm, dst_hbm, out_hbm,
          sidx, didx, msg, stg, acc, semL, semG, semS, semZ, semW):
        zbuf = stg[0]
        cid = lax.axis_index("c")
        sid = lax.axis_index("s")
        wid = sid * NC + cid
        jmax = (nchunk - wid + NW - 1) // NW

        def chunk_valid(j):
            return j < jmax

        def issue_loads(j, b):
            ebase = (wid + j * NW) * CH
            base = eoff + ebase
            pltpu.async_copy(src_hbm.at[pl.ds(base, CH)], sidx[b], semL[b])
            pltpu.async_copy(dst_hbm.at[pl.ds(base, CH)], didx[b], semL[b])
            pltpu.async_copy(e_hbm.at[pl.ds(ebase, CH)], msg[b], semL[b])

        def wait_loads(b):
            pltpu.make_async_copy(src_hbm.at[pl.ds(0, CH)], sidx[b], semL[b]).wait()
            pltpu.make_async_copy(dst_hbm.at[pl.ds(0, CH)], didx[b], semL[b]).wait()
            pltpu.make_async_copy(e_hbm.at[pl.ds(0, CH)], msg[b], semL[b]).wait()

        def zrow(r, carry):
            for j in range(D // LANES):
                zbuf[r, pl.ds(j * LANES, LANES)] = jnp.zeros((LANES,), jnp.float32)
            return carry
        lax.fori_loop(0, ZROWS, zrow, 0)
        for t in range(ZBLK_PER_S):
            b = sid + t * NS

            @pl.when(b < NZBLK)
            def _():
                pltpu.async_copy(zbuf, acc.at[pl.ds(b * ZROWS, ZROWS)], semZ)
        for t in range(ZBLK_PER_S):
            b = sid + t * NS

            @pl.when(b < NZBLK)
            def _():
                pltpu.make_async_copy(
                    zbuf, acc.at[pl.ds(0, ZROWS)], semZ).wait()
        plsc.subcore_barrier()

        issue_loads(0, 0)

        nround = (chunks_per_w + 2 + NB - 1) // NB + 1

        def round_body(r, carry):
            for b in range(NB):
                j = r * NB + b

                @pl.when(chunk_valid(j))
                def _():
                    wait_loads(b)
                    pltpu.async_copy(h_hbm.at[sidx[b]], msg[b], semG[b], add=True)

                bn = (b + 1) % NB

                @pl.when((j >= 2) & chunk_valid(j - 2))
                def _():
                    pltpu.make_async_copy(msg[bn], acc.at[didx[bn]], semS[bn]).wait()

                @pl.when((j + 1 >= 1) & chunk_valid(j + 1))
                def _():
                    issue_loads(j + 1, bn)

                @pl.when(chunk_valid(j))
                def _():
                    pltpu.make_async_copy(h_hbm.at[sidx[b]], msg[b], semG[b]).wait()

                    @plsc.parallel_loop(0, CH, 1, unroll=4)
                    def _relu_row(r):
                        for q in range(D // LANES):
                            sl = pl.ds(q * LANES, LANES)
                            msg[b][r, sl] = jnp.maximum(msg[b][r, sl], 0.0)
                    pltpu.async_copy(msg[b], acc.at[didx[b]], semS[b], add=True)
            return carry
        lax.fori_loop(0, nround, round_body, 0)

        plsc.subcore_barrier()
        for t in range(ZBLK_PER_S):
            b = sid + t * NS
            s = t % 2

            @pl.when(b < NZBLK)
            def _():
                if t >= 2:
                    pltpu.make_async_copy(
                        stg[s], out_hbm.at[cid, pl.ds(0, ZROWS)], semW[s]).wait()
                pltpu.sync_copy(acc.at[pl.ds(b * ZROWS, ZROWS)], stg[s])
                pltpu.async_copy(
                    stg[s], out_hbm.at[cid, pl.ds(b * ZROWS, ZROWS)], semW[s])
        for t in range(max(0, ZBLK_PER_S - 2), ZBLK_PER_S):
            b = sid + t * NS

            @pl.when(b < NZBLK)
            def _():
                pltpu.make_async_copy(
                    stg[t % 2], out_hbm.at[cid, pl.ds(0, ZROWS)],
                    semW[t % 2]).wait()

    return k(h, e, src, dst)


def _mlp(h, parts_list, W1, b1, g, be, W2, b2, relu_out):
    np_ = len(parts_list)

    def body(h_ref, *refs):
        p_refs = refs[:np_]
        w1_ref, b1_ref, g_ref, be_ref, w2_ref, b2_ref, o_ref = refs[np_:]
        r = h_ref[...]
        for p_ref in p_refs:
            r = r + p_ref[0] + p_ref[1]
        t = jnp.dot(r, w1_ref[...], preferred_element_type=jnp.float32) + b1_ref[...]
        m = jnp.mean(t, axis=0, keepdims=True)
        v = jnp.mean((t - m) ** 2, axis=0, keepdims=True)
        t = (t - m) / jnp.sqrt(v + 1e-5) * g_ref[...] + be_ref[...]
        t = jnp.dot(jnp.maximum(t, 0.0), w2_ref[...],
                    preferred_element_type=jnp.float32) + b2_ref[...]
        if relu_out:
            t = jnp.maximum(t, 0.0)
        o_ref[...] = t

    return pl.pallas_call(
        body,
        out_shape=jax.ShapeDtypeStruct((N, D), jnp.float32),
    )(h, *parts_list, W1, b1[None], g[None], be[None], W2, b2[None])


def kernel(nfeat, efeat, params, edge_index):
    src = edge_index[0]
    dst = edge_index[1]
    E2 = E // 2
    (e0a,) = _edge_proj(efeat, params["edge"][0][0][None],
                        params["edge"][0][1][None], 0, E2)
    (e0b,) = _edge_proj(efeat, params["edge"][0][0][None],
                        params["edge"][0][1][None], E2, E2)
    e1, e2 = _edge_proj(
        efeat,
        jnp.stack([params["edge"][1][0], params["edge"][2][0]]),
        jnp.stack([params["edge"][1][1], params["edge"][2][1]]), 0, E)
    h = nfeat
    for i in range(3):
        if i == 0:
            parts = [_sc_edge_agg(h, e0a, src, dst, 0, E2),
                     _sc_edge_agg(h, e0b, src, dst, E2, E2)]
        else:
            parts = [_sc_edge_agg(h, (e1, e2)[i - 1], src, dst, 0, E)]
        p = params["mlp"][i]
        h = _mlp(h, parts, p["W1"], p["b1"], p["g"], p["be"], p["W2"], p["b2"],
                 relu_out=(i != 2))
    return h
